# Initial kernel scaffold; baseline (speedup 1.0000x reference)
import functools

import jax
import jax.numpy as jnp
from jax import lax
from jax.experimental import pallas as pl
from jax.experimental.pallas import tpu as pltpu
from jax.experimental.pallas import tpu_sc as plsc

B = 16384
L = 50
D = 32
LANES = 16

NC, NS = 2, 16
NW = NC * NS
RPB = 2
IPB = RPB * L
NBLK = B // RPB
BPW = NBLK // NW
ROWS_PW = B // NW
RING = 8

_mesh = plsc.VectorSubcoreMesh(
    core_axis_name="c", subcore_axis_name="s", num_cores=NC, num_subcores=NS
)


@functools.partial(
    pl.kernel,
    out_type=jax.ShapeDtypeStruct((B, D), jnp.float32),
    mesh=_mesh,
    scratch_types=[
        pltpu.VMEM((BPW, IPB), jnp.int32),
        pltpu.VMEM((RING, IPB, D), jnp.float32),
        pltpu.VMEM((ROWS_PW, D), jnp.float32),
        pltpu.SemaphoreType.DMA,
    ]
    + [pltpu.SemaphoreType.DMA] * RING,
)
def _sc_sum(idx_hbm, table_hbm, out_hbm, idx_v, rows_v, out_v, sem_i, *sems):
    wid = lax.axis_index("s") * NC + lax.axis_index("c")
    blk0 = wid * BPW

    idx_cp = pltpu.make_async_copy(
        idx_hbm.at[pl.ds(blk0, BPW), :], idx_v, sem_i
    )
    idx_cp.start()
    idx_cp.wait()

    for s in range(RING):
        pltpu.make_async_copy(
            table_hbm.at[idx_v.at[s]], rows_v.at[s], sems[s]
        ).start()

    def body(k, carry):
        for s in range(RING):
            j = k * RING + s
            pltpu.make_async_copy(
                table_hbm.at[idx_v.at[j]], rows_v.at[s], sems[s]
            ).wait()
            for r in range(RPB):
                base = r * L
                a0 = rows_v[s, base, pl.ds(0, LANES)]
                a1 = rows_v[s, base, pl.ds(LANES, LANES)]
                for q in range(1, L):
                    a0 = a0 + rows_v[s, base + q, pl.ds(0, LANES)]
                    a1 = a1 + rows_v[s, base + q, pl.ds(LANES, LANES)]
                orow = j * RPB + r
                out_v[orow, pl.ds(0, LANES)] = a0
                out_v[orow, pl.ds(LANES, LANES)] = a1
            nxt = j + RING

            @pl.when(nxt < BPW)
            def _():
                pltpu.make_async_copy(
                    table_hbm.at[idx_v.at[nxt]], rows_v.at[s], sems[s]
                ).start()

        return carry

    lax.fori_loop(0, BPW // RING, body, 0)

    pltpu.sync_copy(out_v, out_hbm.at[pl.ds(wid * ROWS_PW, ROWS_PW), :])


def _fin_body(idx_ref, sums_ref, t0_ref, out_ref):
    idx = idx_ref[...]
    sums = sums_ref[...]
    t0 = t0_ref[...]
    cnt = jnp.sum((idx != 0).astype(jnp.float32), axis=1, keepdims=True)
    nz = jnp.float32(L) - cnt
    mean = (sums - nz * t0) / cnt
    nrm = jnp.sqrt(jnp.sum(mean * mean, axis=1, keepdims=True))
    out_ref[...] = mean / jnp.maximum(nrm, 1e-12)


_FIN_BLK = 1024

_fin = pl.pallas_call(
    _fin_body,
    grid=(B // _FIN_BLK,),
    in_specs=[
        pl.BlockSpec((_FIN_BLK, L), lambda i: (i, 0)),
        pl.BlockSpec((_FIN_BLK, D), lambda i: (i, 0)),
        pl.BlockSpec((1, D), lambda i: (0, 0)),
    ],
    out_specs=pl.BlockSpec((_FIN_BLK, D), lambda i: (i, 0)),
    out_shape=jax.ShapeDtypeStruct((B, D), jnp.float32),
)


@jax.jit
def kernel(ingredient_indices, table):
    idx2d = ingredient_indices.reshape(NBLK, IPB)
    sums = _sc_sum(idx2d, table)
    return _fin(ingredient_indices, sums, table[0:1])


# speedup vs baseline: 2.7083x; 2.7083x over previous
import functools

import jax
import jax.numpy as jnp
from jax import lax
from jax.experimental import pallas as pl
from jax.experimental.pallas import tpu as pltpu
from jax.experimental.pallas import tpu_sc as plsc

B = 16384
L = 50
D = 32
LANES = 16

NC, NS = 2, 16
NW = NC * NS
RPB = 2
IPB = RPB * L
NBLK = B // RPB
BPW = NBLK // NW
ROWS_PW = B // NW
RING = 8

_mesh = plsc.VectorSubcoreMesh(
    core_axis_name="c", subcore_axis_name="s", num_cores=NC, num_subcores=NS
)


@functools.partial(
    pl.kernel,
    out_type=jax.ShapeDtypeStruct((B, D), jnp.float32),
    mesh=_mesh,
    scratch_types=[
        pltpu.VMEM((BPW, IPB), jnp.int32),
        pltpu.VMEM((RING, IPB, D), jnp.float32),
        pltpu.VMEM((ROWS_PW, D), jnp.float32),
        pltpu.SemaphoreType.DMA,
    ]
    + [pltpu.SemaphoreType.DMA] * RING,
    compiler_params=pltpu.CompilerParams(use_tc_tiling_on_sc=False),
)
def _sc_sum(idx_hbm, table_hbm, out_hbm, idx_v, rows_v, out_v, sem_i, *sems):
    wid = lax.axis_index("s") * NC + lax.axis_index("c")
    blk0 = wid * BPW

    idx_cp = pltpu.make_async_copy(
        idx_hbm.at[pl.ds(blk0, BPW), :], idx_v, sem_i
    )
    idx_cp.start()
    idx_cp.wait()

    for s in range(RING):
        pltpu.make_async_copy(
            table_hbm.at[idx_v.at[s]], rows_v.at[s], sems[s]
        ).start()

    def body(k, carry):
        for s in range(RING):
            j = k * RING + s
            pltpu.make_async_copy(
                table_hbm.at[idx_v.at[j]], rows_v.at[s], sems[s]
            ).wait()
            for r in range(RPB):
                base = r * L
                a0 = rows_v[s, base, pl.ds(0, LANES)]
                a1 = rows_v[s, base, pl.ds(LANES, LANES)]
                for q in range(1, L):
                    a0 = a0 + rows_v[s, base + q, pl.ds(0, LANES)]
                    a1 = a1 + rows_v[s, base + q, pl.ds(LANES, LANES)]
                orow = j * RPB + r
                out_v[orow, pl.ds(0, LANES)] = a0
                out_v[orow, pl.ds(LANES, LANES)] = a1
            nxt = j + RING

            @pl.when(nxt < BPW)
            def _():
                pltpu.make_async_copy(
                    table_hbm.at[idx_v.at[nxt]], rows_v.at[s], sems[s]
                ).start()

        return carry

    lax.fori_loop(0, BPW // RING, body, 0)

    pltpu.sync_copy(out_v, out_hbm.at[pl.ds(wid * ROWS_PW, ROWS_PW), :])


def _fin_body(idx_ref, sums_ref, t0_ref, out_ref):
    idx = idx_ref[...]
    sums = sums_ref[...]
    t0 = t0_ref[...]
    cnt = jnp.sum((idx != 0).astype(jnp.float32), axis=1, keepdims=True)
    nz = jnp.float32(L) - cnt
    mean = (sums - nz * t0) / cnt
    nrm = jnp.sqrt(jnp.sum(mean * mean, axis=1, keepdims=True))
    out_ref[...] = mean / jnp.maximum(nrm, 1e-12)


_FIN_BLK = 1024

_fin = pl.pallas_call(
    _fin_body,
    grid=(B // _FIN_BLK,),
    in_specs=[
        pl.BlockSpec((_FIN_BLK, L), lambda i: (i, 0)),
        pl.BlockSpec((_FIN_BLK, D), lambda i: (i, 0)),
        pl.BlockSpec((1, D), lambda i: (0, 0)),
    ],
    out_specs=pl.BlockSpec((_FIN_BLK, D), lambda i: (i, 0)),
    out_shape=jax.ShapeDtypeStruct((B, D), jnp.float32),
)


@jax.jit
def kernel(ingredient_indices, table):
    idx2d = ingredient_indices.reshape(NBLK, IPB)
    sums = _sc_sum(idx2d, table)
    return _fin(ingredient_indices, sums, table[0:1])
